# baseline (speedup 1.0000x reference)
import functools

import jax
import jax.numpy as jnp
from jax import lax
from jax.experimental import pallas as pl
from jax.experimental.pallas import tpu as pltpu
from jax.experimental.pallas import tpu_sc as plsc

S = 2048
H = 768
I = 768
E = 8
K = 2
ALPHA = 1.702
LIMIT = 7.0

BLK = 128
CAP = S * K + E * BLK
NB = CAP // BLK
LANES = 128

NC = 2
NS = 16
NW = NC * NS
SC_L = 16

NEG = -1e30


def _router_body(x_ref, w_ref, b_ref, scores_ref, dst_ref, be_ref, ohs_ref, t_ref):
    x = x_ref[...]
    w = w_ref[...]
    logits = jnp.dot(x, w, preferred_element_type=jnp.float32) + b_ref[...]
    lane = lax.broadcasted_iota(jnp.int32, (S, LANES), 1)
    valid = lane < E
    logits = jnp.where(valid, logits, NEG)

    m1 = jnp.max(logits, axis=1, keepdims=True)
    i1 = jnp.min(jnp.where(logits == m1, lane, LANES), axis=1, keepdims=True)
    l2 = jnp.where(lane == i1, NEG, logits)
    m2 = jnp.max(l2, axis=1, keepdims=True)
    i2 = jnp.min(jnp.where(l2 == m2, lane, LANES), axis=1, keepdims=True)

    t = jnp.exp(m2 - m1)
    s1 = 1.0 / (1.0 + t)
    s2 = t / (1.0 + t)
    scores_ref[...] = jnp.where(lane == 0, s1, jnp.where(lane == 1, s2, 0.0))

    onehot0 = (lane == i1).astype(jnp.float32)
    onehot1 = (lane == i2).astype(jnp.float32)
    ohs_ref[...] = onehot0 + onehot1

    r_sub = lax.broadcasted_iota(jnp.int32, (BLK, BLK), 0)
    r_lane = lax.broadcasted_iota(jnp.int32, (BLK, BLK), 1)
    ltri = (r_sub >= r_lane).astype(jnp.float32)

    def chunk_step(c, carry):
        chunk = ohs_ref[pl.ds(c * BLK, BLK), :]
        tc = jnp.dot(ltri, chunk, preferred_element_type=jnp.float32) + carry
        t_ref[pl.ds(c * BLK, BLK), :] = tc
        return tc[BLK - 1 : BLK, :]

    carry0 = jnp.zeros((1, LANES), jnp.float32)
    lax.fori_loop(0, S // BLK, chunk_step, carry0)

    counts = t_ref[S - 1 : S, :]
    aligned = jnp.floor((counts + (BLK - 1)) / BLK) * BLK
    stri = (r_sub < r_lane).astype(jnp.float32)
    off = jnp.dot(aligned, stri, preferred_element_type=jnp.float32)
    ends = off + aligned

    tfull = t_ref[...]
    dst0 = jnp.sum(onehot0 * (off + tfull), axis=1, keepdims=True) - 1.0
    dst1 = jnp.sum(onehot1 * (off + tfull), axis=1, keepdims=True) - 1.0
    dsts = jnp.where(lane == 0, dst0, jnp.where(lane == 1, dst1, 0.0))
    dst_ref[...] = dsts.astype(jnp.int32)

    v_sub = lax.broadcasted_iota(jnp.int32, (NB, LANES), 0).astype(jnp.float32) * BLK
    be_lane = lax.broadcasted_iota(jnp.int32, (NB, LANES), 1)
    ind = ((v_sub >= ends) & (be_lane < E)).astype(jnp.int32)
    be = jnp.minimum(jnp.sum(ind, axis=1, keepdims=True), E - 1)
    be_ref[...] = jnp.broadcast_to(be, (NB, LANES))


def _stage1(x, w_pad, b_pad):
    return pl.pallas_call(
        _router_body,
        out_shape=(
            jax.ShapeDtypeStruct((S, LANES), jnp.float32),
            jax.ShapeDtypeStruct((S, LANES), jnp.int32),
            jax.ShapeDtypeStruct((NB, LANES), jnp.int32),
        ),
        scratch_shapes=[
            pltpu.VMEM((S, LANES), jnp.float32),
            pltpu.VMEM((S, LANES), jnp.float32),
        ],
    )(x, w_pad, b_pad)


_SLOTS_PW = CAP // NW
_PAIRS = S * K
_ROWS_PW = CAP // NW
_GCHUNK = 80
_TOK_PW = S // NW


@functools.cache
def _sc_stages():
    mesh = plsc.VectorSubcoreMesh(
        core_axis_name="c", subcore_axis_name="s", num_cores=NC, num_subcores=NS
    )

    @functools.partial(
        pl.kernel,
        out_type=jax.ShapeDtypeStruct((CAP,), jnp.int32),
        mesh=mesh,
        compiler_params=pltpu.CompilerParams(needs_layout_passes=False),
        scratch_types=[
            pltpu.VMEM((_PAIRS,), jnp.int32),
            pltpu.VMEM((2 * BLK,), jnp.int32),
        ],
    )
    def stage2(dst_hbm, out_hbm, dst_v, st_v):
        wid = lax.axis_index("s") * NC + lax.axis_index("c")
        base = wid * _SLOTS_PW
        pltpu.sync_copy(dst_hbm, dst_v)

        for z in range(_SLOTS_PW // SC_L):
            st_v[pl.ds(z * SC_L, SC_L)] = jnp.zeros((SC_L,), jnp.int32)

        def step(c, _):
            idx = dst_v[pl.ds(c * SC_L, SC_L)]
            tok = (c * SC_L + lax.iota(jnp.int32, SC_L)) >> 1
            rel = idx - base
            mask = (rel >= 0) & (rel < _SLOTS_PW)
            relc = jnp.where(mask, rel, 0)
            plsc.store_scatter(st_v, [relc], tok, mask=mask)
            return 0

        lax.fori_loop(0, _PAIRS // SC_L, step, 0)
        pltpu.sync_copy(st_v.at[pl.ds(0, _SLOTS_PW)],
                        out_hbm.at[pl.ds(base, _SLOTS_PW)])

    @functools.partial(
        pl.kernel,
        out_type=jax.ShapeDtypeStruct((CAP, H // 2), jnp.int32),
        mesh=mesh,
        compiler_params=pltpu.CompilerParams(needs_layout_passes=False),
        scratch_types=[
            pltpu.VMEM((_ROWS_PW,), jnp.int32),
            pltpu.VMEM((_GCHUNK, H // 2), jnp.int32),
            pltpu.VMEM((_GCHUNK, H // 2), jnp.int32),
            pltpu.SemaphoreType.DMA,
        ],
    )
    def stage3(x_hbm, srctok_hbm, out_hbm, idx_v, rows0_v, rows1_v, sem):
        wid = lax.axis_index("s") * NC + lax.axis_index("c")
        base = wid * _ROWS_PW
        pltpu.sync_copy(srctok_hbm.at[pl.ds(base, _ROWS_PW)], idx_v)
        h0 = pltpu.async_copy(x_hbm.at[idx_v.at[pl.ds(0, _GCHUNK)]], rows0_v, sem)
        h1 = pltpu.async_copy(x_hbm.at[idx_v.at[pl.ds(_GCHUNK, _GCHUNK)]], rows1_v, sem)
        h0.wait()
        pltpu.sync_copy(rows0_v, out_hbm.at[pl.ds(base, _GCHUNK)])
        h1.wait()
        pltpu.sync_copy(rows1_v, out_hbm.at[pl.ds(base + _GCHUNK, _GCHUNK)])

    @functools.partial(
        pl.kernel,
        out_type=jax.ShapeDtypeStruct((S, H), jnp.float32),
        mesh=mesh,
        compiler_params=pltpu.CompilerParams(needs_layout_passes=False),
        scratch_types=[
            pltpu.VMEM((2 * _TOK_PW,), jnp.int32),
            pltpu.VMEM((2 * _TOK_PW,), jnp.float32),
            pltpu.VMEM((_TOK_PW,), jnp.int32),
            pltpu.VMEM((_TOK_PW,), jnp.int32),
            pltpu.VMEM((_TOK_PW, H), jnp.float32),
            pltpu.VMEM((_TOK_PW, H), jnp.float32),
            pltpu.SemaphoreType.DMA,
        ],
    )
    def stage5(y_hbm, dst_hbm, sc_hbm, out_hbm, dst_v, sc_v, i0_v, i1_v,
               b0_v, b1_v, sem):
        wid = lax.axis_index("s") * NC + lax.axis_index("c")
        base = wid * _TOK_PW
        pltpu.sync_copy(dst_hbm.at[pl.ds(base * 2, 2 * _TOK_PW)], dst_v)
        pltpu.sync_copy(sc_hbm.at[pl.ds(base * 2, 2 * _TOK_PW)], sc_v)

        for z in range(_TOK_PW // SC_L):
            g = (z * SC_L + lax.iota(jnp.int32, SC_L)) * 2
            i0_v[pl.ds(z * SC_L, SC_L)] = plsc.load_gather(dst_v, [g])
            i1_v[pl.ds(z * SC_L, SC_L)] = plsc.load_gather(dst_v, [g + 1])

        pltpu.async_copy(y_hbm.at[i0_v], b0_v, sem).wait()
        pltpu.async_copy(y_hbm.at[i1_v], b1_v, sem).wait()

        def row_step(i, _):
            s0 = plsc.load_gather(sc_v, [jnp.full((SC_L,), 2 * i, jnp.int32)])
            s1 = plsc.load_gather(sc_v, [jnp.full((SC_L,), 2 * i + 1, jnp.int32)])
            for h in range(H // SC_L):
                sl = pl.ds(h * SC_L, SC_L)
                b0_v[i, sl] = s0 * b0_v[i, sl] + s1 * b1_v[i, sl]
            return 0

        lax.fori_loop(0, _TOK_PW, row_step, 0)
        pltpu.sync_copy(b0_v, out_hbm.at[pl.ds(base, _TOK_PW)])

    return stage2, stage3, stage5


def _mlp_body(be_ref, x_ref, wgu_ref, bgu_ref, wd_ref, bd_ref, y_ref):
    x = x_ref[...]
    gu = jnp.dot(x, wgu_ref[0], preferred_element_type=jnp.float32)
    gu = gu + bgu_ref[0]
    gu = jnp.minimum(gu, LIMIT)
    gate = gu[:, :I]
    up = jnp.maximum(gu[:, I:], -LIMIT)
    glu = gate * jax.nn.sigmoid(gate * ALPHA)
    act = ((up + 1.0) * glu).astype(jnp.bfloat16)
    y = jnp.dot(act, wd_ref[0], preferred_element_type=jnp.float32)
    y_ref[...] = y + bd_ref[0]


def _stage4(be, xs, wgu, bgu, wd, bd):
    grid_spec = pltpu.PrefetchScalarGridSpec(
        num_scalar_prefetch=1,
        grid=(NB,),
        in_specs=[
            pl.BlockSpec((BLK, H), lambda v, be: (v, 0)),
            pl.BlockSpec((1, H, 2 * I), lambda v, be: (be[v], 0, 0)),
            pl.BlockSpec((1, 1, 2 * I), lambda v, be: (be[v], 0, 0)),
            pl.BlockSpec((1, I, H), lambda v, be: (be[v], 0, 0)),
            pl.BlockSpec((1, 1, H), lambda v, be: (be[v], 0, 0)),
        ],
        out_specs=pl.BlockSpec((BLK, H), lambda v, be: (v, 0)),
    )
    return pl.pallas_call(
        _mlp_body,
        grid_spec=grid_spec,
        out_shape=jax.ShapeDtypeStruct((CAP, H), jnp.float32),
    )(be, xs, wgu, bgu.reshape(E, 1, 2 * I), wd, bd.reshape(E, 1, H))


def kernel(hidden_states, router_w, router_b, gate_up_proj, gate_up_proj_bias,
           down_proj, down_proj_bias):
    B = hidden_states.shape[0]
    x2 = hidden_states.reshape(S, H)
    w_pad = jnp.pad(router_w, ((0, 0), (0, LANES - E)))
    b_pad = jnp.pad(router_b, (0, LANES - E)).reshape(1, LANES)

    scores_pad, dst_pad, be_pad = _stage1(x2, w_pad, b_pad)
    scores2 = scores_pad[:, :K]
    dst2 = dst_pad[:, :K]
    be = be_pad[:, 0]
    dst_flat = dst2.reshape(_PAIRS)

    stage2, stage3, stage5 = _sc_stages()
    src_tok = stage2(dst_flat)
    x_i32 = lax.bitcast_convert_type(
        hidden_states.astype(jnp.bfloat16).reshape(S, H // 2, 2), jnp.int32)
    xs_i32 = stage3(x_i32, src_tok)
    xs = lax.bitcast_convert_type(xs_i32, jnp.bfloat16).reshape(CAP, H)
    ys = _stage4(be, xs, gate_up_proj.astype(jnp.bfloat16), gate_up_proj_bias,
                 down_proj.astype(jnp.bfloat16), down_proj_bias)
    out = stage5(ys, dst_flat, scores2.reshape(_PAIRS))

    return out.reshape(B, S, H), scores2.reshape(B, S, K)


# speedup vs baseline: 1.2725x; 1.2725x over previous
import functools

import jax
import jax.numpy as jnp
from jax import lax
from jax.experimental import pallas as pl
from jax.experimental.pallas import tpu as pltpu
from jax.experimental.pallas import tpu_sc as plsc

S = 2048
H = 768
I = 768
E = 8
K = 2
ALPHA = 1.702
LIMIT = 7.0

BLK = 128
CAP = S * K + E * BLK
NB = CAP // BLK
LANES = 128

NC = 2
NS = 16
NW = NC * NS
SC_L = 16

NEG = -1e30


def _router_body(x_ref, w_ref, b_ref, scores_ref, dst_ref, be_ref, ohs_ref, t_ref):
    x = x_ref[...]
    w = w_ref[...]
    logits = jnp.dot(x, w, preferred_element_type=jnp.float32) + b_ref[...]
    lane = lax.broadcasted_iota(jnp.int32, (S, LANES), 1)
    valid = lane < E
    logits = jnp.where(valid, logits, NEG)

    m1 = jnp.max(logits, axis=1, keepdims=True)
    i1 = jnp.min(jnp.where(logits == m1, lane, LANES), axis=1, keepdims=True)
    l2 = jnp.where(lane == i1, NEG, logits)
    m2 = jnp.max(l2, axis=1, keepdims=True)
    i2 = jnp.min(jnp.where(l2 == m2, lane, LANES), axis=1, keepdims=True)

    t = jnp.exp(m2 - m1)
    s1 = 1.0 / (1.0 + t)
    s2 = t / (1.0 + t)
    scores_ref[...] = jnp.where(lane == 0, s1, jnp.where(lane == 1, s2, 0.0))

    onehot0 = (lane == i1).astype(jnp.float32)
    onehot1 = (lane == i2).astype(jnp.float32)
    ohs_ref[...] = onehot0 + onehot1

    r_sub = lax.broadcasted_iota(jnp.int32, (BLK, BLK), 0)
    r_lane = lax.broadcasted_iota(jnp.int32, (BLK, BLK), 1)
    ltri = (r_sub >= r_lane).astype(jnp.float32)

    def chunk_step(c, carry):
        chunk = ohs_ref[pl.ds(c * BLK, BLK), :]
        tc = jnp.dot(ltri, chunk, preferred_element_type=jnp.float32) + carry
        t_ref[pl.ds(c * BLK, BLK), :] = tc
        return tc[BLK - 1 : BLK, :]

    carry0 = jnp.zeros((1, LANES), jnp.float32)
    lax.fori_loop(0, S // BLK, chunk_step, carry0)

    counts = t_ref[S - 1 : S, :]
    aligned = jnp.floor((counts + (BLK - 1)) / BLK) * BLK
    stri = (r_sub < r_lane).astype(jnp.float32)
    off = jnp.dot(aligned, stri, preferred_element_type=jnp.float32)
    ends = off + aligned

    tfull = t_ref[...]
    dst0 = jnp.sum(onehot0 * (off + tfull), axis=1, keepdims=True) - 1.0
    dst1 = jnp.sum(onehot1 * (off + tfull), axis=1, keepdims=True) - 1.0
    dsts = jnp.where(lane == 0, dst0, jnp.where(lane == 1, dst1, 0.0))
    dst_ref[...] = dsts.astype(jnp.int32)

    v_sub = lax.broadcasted_iota(jnp.int32, (NB, LANES), 0).astype(jnp.float32) * BLK
    be_lane = lax.broadcasted_iota(jnp.int32, (NB, LANES), 1)
    ind = ((v_sub >= ends) & (be_lane < E)).astype(jnp.int32)
    be = jnp.minimum(jnp.sum(ind, axis=1, keepdims=True), E - 1)
    be_ref[...] = jnp.broadcast_to(be, (NB, LANES))


def _stage1(x, w_pad, b_pad):
    return pl.pallas_call(
        _router_body,
        out_shape=(
            jax.ShapeDtypeStruct((S, LANES), jnp.float32),
            jax.ShapeDtypeStruct((S, LANES), jnp.int32),
            jax.ShapeDtypeStruct((NB, LANES), jnp.int32),
        ),
        scratch_shapes=[
            pltpu.VMEM((S, LANES), jnp.float32),
            pltpu.VMEM((S, LANES), jnp.float32),
        ],
    )(x, w_pad, b_pad)


_SLOTS_PW = CAP // NW
_PAIRS = S * K
_ROWS_PW = CAP // NW
_GCHUNK = 80
_TOK_PW = S // NW


@functools.cache
def _sc_stages():
    mesh = plsc.VectorSubcoreMesh(
        core_axis_name="c", subcore_axis_name="s", num_cores=NC, num_subcores=NS
    )

    @functools.partial(
        pl.kernel,
        out_type=jax.ShapeDtypeStruct((CAP,), jnp.int32),
        mesh=mesh,
        compiler_params=pltpu.CompilerParams(needs_layout_passes=False),
        scratch_types=[
            pltpu.VMEM((_PAIRS,), jnp.int32),
            pltpu.VMEM((2 * BLK,), jnp.int32),
        ],
    )
    def stage2(dst_hbm, out_hbm, dst_v, st_v):
        wid = lax.axis_index("s") * NC + lax.axis_index("c")
        base = wid * _SLOTS_PW
        pltpu.sync_copy(dst_hbm, dst_v)

        for z in range(_SLOTS_PW // SC_L):
            st_v[pl.ds(z * SC_L, SC_L)] = jnp.zeros((SC_L,), jnp.int32)

        def step(c, _):
            idx = dst_v[pl.ds(c * SC_L, SC_L)]
            tok = (c * SC_L + lax.iota(jnp.int32, SC_L)) >> 1
            rel = idx - base
            mask = (rel >= 0) & (rel < _SLOTS_PW)
            relc = jnp.where(mask, rel, 0)
            plsc.store_scatter(st_v, [relc], tok, mask=mask)
            return 0

        lax.fori_loop(0, _PAIRS // SC_L, step, 0)
        pltpu.sync_copy(st_v.at[pl.ds(0, _SLOTS_PW)],
                        out_hbm.at[pl.ds(base, _SLOTS_PW)])

    @functools.partial(
        pl.kernel,
        out_type=jax.ShapeDtypeStruct((CAP, H), jnp.bfloat16),
        mesh=mesh,
        compiler_params=pltpu.CompilerParams(
            needs_layout_passes=False, use_tc_tiling_on_sc=False),
        scratch_types=[
            pltpu.VMEM((_ROWS_PW,), jnp.int32),
            pltpu.VMEM((_GCHUNK, H), jnp.bfloat16),
            pltpu.VMEM((_GCHUNK, H), jnp.bfloat16),
            pltpu.SemaphoreType.DMA,
        ],
    )
    def stage3(x_hbm, srctok_hbm, out_hbm, idx_v, rows0_v, rows1_v, sem):
        wid = lax.axis_index("s") * NC + lax.axis_index("c")
        base = wid * _ROWS_PW
        pltpu.sync_copy(srctok_hbm.at[pl.ds(base, _ROWS_PW)], idx_v)
        h0 = pltpu.async_copy(x_hbm.at[idx_v.at[pl.ds(0, _GCHUNK)]], rows0_v, sem)
        h1 = pltpu.async_copy(x_hbm.at[idx_v.at[pl.ds(_GCHUNK, _GCHUNK)]], rows1_v, sem)
        h0.wait()
        pltpu.sync_copy(rows0_v, out_hbm.at[pl.ds(base, _GCHUNK)])
        h1.wait()
        pltpu.sync_copy(rows1_v, out_hbm.at[pl.ds(base + _GCHUNK, _GCHUNK)])

    @functools.partial(
        pl.kernel,
        out_type=jax.ShapeDtypeStruct((S, H), jnp.float32),
        mesh=mesh,
        compiler_params=pltpu.CompilerParams(needs_layout_passes=False),
        scratch_types=[
            pltpu.VMEM((2 * _TOK_PW,), jnp.int32),
            pltpu.VMEM((2 * _TOK_PW,), jnp.float32),
            pltpu.VMEM((_TOK_PW,), jnp.int32),
            pltpu.VMEM((_TOK_PW,), jnp.int32),
            pltpu.VMEM((_TOK_PW, H), jnp.float32),
            pltpu.VMEM((_TOK_PW, H), jnp.float32),
            pltpu.SemaphoreType.DMA,
        ],
    )
    def stage5(y_hbm, dst_hbm, sc_hbm, out_hbm, dst_v, sc_v, i0_v, i1_v,
               b0_v, b1_v, sem):
        wid = lax.axis_index("s") * NC + lax.axis_index("c")
        base = wid * _TOK_PW
        pltpu.sync_copy(dst_hbm.at[pl.ds(base * 2, 2 * _TOK_PW)], dst_v)
        pltpu.sync_copy(sc_hbm.at[pl.ds(base * 2, 2 * _TOK_PW)], sc_v)

        for z in range(_TOK_PW // SC_L):
            g = (z * SC_L + lax.iota(jnp.int32, SC_L)) * 2
            i0_v[pl.ds(z * SC_L, SC_L)] = plsc.load_gather(dst_v, [g])
            i1_v[pl.ds(z * SC_L, SC_L)] = plsc.load_gather(dst_v, [g + 1])

        pltpu.async_copy(y_hbm.at[i0_v], b0_v, sem).wait()
        pltpu.async_copy(y_hbm.at[i1_v], b1_v, sem).wait()

        def row_step(i, _):
            s0 = plsc.load_gather(sc_v, [jnp.full((SC_L,), 2 * i, jnp.int32)])
            s1 = plsc.load_gather(sc_v, [jnp.full((SC_L,), 2 * i + 1, jnp.int32)])
            for h in range(H // SC_L):
                sl = pl.ds(h * SC_L, SC_L)
                b0_v[i, sl] = s0 * b0_v[i, sl] + s1 * b1_v[i, sl]
            return 0

        lax.fori_loop(0, _TOK_PW, row_step, 0)
        pltpu.sync_copy(b0_v, out_hbm.at[pl.ds(base, _TOK_PW)])

    return stage2, stage3, stage5


def _mlp_body(be_ref, x_ref, wgu_ref, bgu_ref, wd_ref, bd_ref, y_ref):
    x = x_ref[...]
    gu = jnp.dot(x, wgu_ref[0], preferred_element_type=jnp.float32)
    gu = gu + bgu_ref[0]
    gu = jnp.minimum(gu, LIMIT)
    gate = gu[:, :I]
    up = jnp.maximum(gu[:, I:], -LIMIT)
    glu = gate * jax.nn.sigmoid(gate * ALPHA)
    act = ((up + 1.0) * glu).astype(jnp.bfloat16)
    y = jnp.dot(act, wd_ref[0], preferred_element_type=jnp.float32)
    y_ref[...] = y + bd_ref[0]


def _stage4(be, xs, wgu, bgu, wd, bd):
    grid_spec = pltpu.PrefetchScalarGridSpec(
        num_scalar_prefetch=1,
        grid=(NB,),
        in_specs=[
            pl.BlockSpec((BLK, H), lambda v, be: (v, 0)),
            pl.BlockSpec((1, H, 2 * I), lambda v, be: (be[v], 0, 0)),
            pl.BlockSpec((1, 1, 2 * I), lambda v, be: (be[v], 0, 0)),
            pl.BlockSpec((1, I, H), lambda v, be: (be[v], 0, 0)),
            pl.BlockSpec((1, 1, H), lambda v, be: (be[v], 0, 0)),
        ],
        out_specs=pl.BlockSpec((BLK, H), lambda v, be: (v, 0)),
    )
    return pl.pallas_call(
        _mlp_body,
        grid_spec=grid_spec,
        out_shape=jax.ShapeDtypeStruct((CAP, H), jnp.float32),
    )(be, xs, wgu, bgu.reshape(E, 1, 2 * I), wd, bd.reshape(E, 1, H))


def kernel(hidden_states, router_w, router_b, gate_up_proj, gate_up_proj_bias,
           down_proj, down_proj_bias):
    B = hidden_states.shape[0]
    x2 = hidden_states.reshape(S, H)
    w_pad = jnp.pad(router_w, ((0, 0), (0, LANES - E)))
    b_pad = jnp.pad(router_b, (0, LANES - E)).reshape(1, LANES)

    scores_pad, dst_pad, be_pad = _stage1(x2, w_pad, b_pad)
    scores2 = scores_pad[:, :K]
    dst2 = dst_pad[:, :K]
    be = be_pad[:, 0]
    dst_flat = dst2.reshape(_PAIRS)

    stage2, stage3, stage5 = _sc_stages()
    src_tok = stage2(dst_flat)
    x_bf = hidden_states.astype(jnp.bfloat16).reshape(S, H)
    xs = stage3(x_bf, src_tok)
    ys = _stage4(be, xs, gate_up_proj.astype(jnp.bfloat16), gate_up_proj_bias,
                 down_proj.astype(jnp.bfloat16), down_proj_bias)
    out = stage5(ys, dst_flat, scores2.reshape(_PAIRS))

    return out.reshape(B, S, H), scores2.reshape(B, S, K)


# speedup vs baseline: 1.3948x; 1.0961x over previous
import functools

import jax
import jax.numpy as jnp
from jax import lax
from jax.experimental import pallas as pl
from jax.experimental.pallas import tpu as pltpu
from jax.experimental.pallas import tpu_sc as plsc

S = 2048
H = 768
I = 768
E = 8
K = 2
ALPHA = 1.702
LIMIT = 7.0

BLK = 128
CAP = S * K + E * BLK
NB = CAP // BLK
LANES = 128

NC = 2
NS = 16
NW = NC * NS
SC_L = 16

NEG = -1e30


def _router_body(x_ref, w_ref, b_ref, scores_ref, dst_ref, be_ref, ohs_ref, t_ref):
    x = x_ref[...]
    w = w_ref[...]
    logits = jnp.dot(x, w, preferred_element_type=jnp.float32) + b_ref[...]
    lane = lax.broadcasted_iota(jnp.int32, (S, LANES), 1)
    valid = lane < E
    logits = jnp.where(valid, logits, NEG)

    m1 = jnp.max(logits, axis=1, keepdims=True)
    i1 = jnp.min(jnp.where(logits == m1, lane, LANES), axis=1, keepdims=True)
    l2 = jnp.where(lane == i1, NEG, logits)
    m2 = jnp.max(l2, axis=1, keepdims=True)
    i2 = jnp.min(jnp.where(l2 == m2, lane, LANES), axis=1, keepdims=True)

    t = jnp.exp(m2 - m1)
    s1 = 1.0 / (1.0 + t)
    s2 = t / (1.0 + t)
    scores_ref[...] = jnp.where(lane == 0, s1, jnp.where(lane == 1, s2, 0.0))

    onehot0 = (lane == i1).astype(jnp.float32)
    onehot1 = (lane == i2).astype(jnp.float32)
    ohs_ref[...] = onehot0 + onehot1

    r_sub = lax.broadcasted_iota(jnp.int32, (BLK, BLK), 0)
    r_lane = lax.broadcasted_iota(jnp.int32, (BLK, BLK), 1)
    ltri = (r_sub >= r_lane).astype(jnp.float32)

    def chunk_step(c, carry):
        chunk = ohs_ref[pl.ds(c * BLK, BLK), :]
        tc = jnp.dot(ltri, chunk, preferred_element_type=jnp.float32) + carry
        t_ref[pl.ds(c * BLK, BLK), :] = tc
        return tc[BLK - 1 : BLK, :]

    carry0 = jnp.zeros((1, LANES), jnp.float32)
    lax.fori_loop(0, S // BLK, chunk_step, carry0)

    counts = t_ref[S - 1 : S, :]
    aligned = jnp.floor((counts + (BLK - 1)) / BLK) * BLK
    stri = (r_sub < r_lane).astype(jnp.float32)
    off = jnp.dot(aligned, stri, preferred_element_type=jnp.float32)
    ends = off + aligned

    tfull = t_ref[...]
    dst0 = jnp.sum(onehot0 * (off + tfull), axis=1, keepdims=True) - 1.0
    dst1 = jnp.sum(onehot1 * (off + tfull), axis=1, keepdims=True) - 1.0
    dsts = jnp.where(lane == 0, dst0, jnp.where(lane == 1, dst1, 0.0))
    dst_ref[...] = dsts.astype(jnp.int32)

    v_sub = lax.broadcasted_iota(jnp.int32, (NB, LANES), 0).astype(jnp.float32) * BLK
    be_lane = lax.broadcasted_iota(jnp.int32, (NB, LANES), 1)
    ind = ((v_sub >= ends) & (be_lane < E)).astype(jnp.int32)
    be = jnp.minimum(jnp.sum(ind, axis=1, keepdims=True), E - 1)
    be_ref[...] = jnp.broadcast_to(be, (NB, LANES))


def _stage1(x, w_pad, b_pad):
    return pl.pallas_call(
        _router_body,
        out_shape=(
            jax.ShapeDtypeStruct((S, LANES), jnp.float32),
            jax.ShapeDtypeStruct((S, LANES), jnp.int32),
            jax.ShapeDtypeStruct((NB, LANES), jnp.int32),
        ),
        scratch_shapes=[
            pltpu.VMEM((S, LANES), jnp.float32),
            pltpu.VMEM((S, LANES), jnp.float32),
        ],
    )(x, w_pad, b_pad)


_SLOTS_PW = CAP // NW
_PAIRS = S * K
_ROWS_PW = CAP // NW
_GCHUNK = 80
_TOK_PW = S // NW


@functools.cache
def _sc_stages():
    mesh = plsc.VectorSubcoreMesh(
        core_axis_name="c", subcore_axis_name="s", num_cores=NC, num_subcores=NS
    )

    @functools.partial(
        pl.kernel,
        out_type=jax.ShapeDtypeStruct((CAP,), jnp.int32),
        mesh=mesh,
        compiler_params=pltpu.CompilerParams(needs_layout_passes=False),
        scratch_types=[
            pltpu.VMEM((_PAIRS,), jnp.int32),
            pltpu.VMEM((2 * BLK,), jnp.int32),
        ],
    )
    def stage2(dst_hbm, out_hbm, dst_v, st_v):
        wid = lax.axis_index("s") * NC + lax.axis_index("c")
        base = wid * _SLOTS_PW
        pltpu.sync_copy(dst_hbm, dst_v)

        for z in range(_SLOTS_PW // SC_L):
            st_v[pl.ds(z * SC_L, SC_L)] = jnp.zeros((SC_L,), jnp.int32)

        def step(c, _):
            idx = dst_v[pl.ds(c * SC_L, SC_L)]
            tok = (c * SC_L + lax.iota(jnp.int32, SC_L)) >> 1
            rel = idx - base
            mask = (rel >= 0) & (rel < _SLOTS_PW)
            relc = jnp.where(mask, rel, 0)
            plsc.store_scatter(st_v, [relc], tok, mask=mask)
            return 0

        lax.fori_loop(0, _PAIRS // SC_L, step, 0)
        pltpu.sync_copy(st_v.at[pl.ds(0, _SLOTS_PW)],
                        out_hbm.at[pl.ds(base, _SLOTS_PW)])

    @functools.partial(
        pl.kernel,
        out_type=jax.ShapeDtypeStruct((CAP, H), jnp.bfloat16),
        mesh=mesh,
        compiler_params=pltpu.CompilerParams(
            needs_layout_passes=False, use_tc_tiling_on_sc=False),
        scratch_types=[
            pltpu.VMEM((_ROWS_PW,), jnp.int32),
            pltpu.VMEM((_GCHUNK, H), jnp.bfloat16),
            pltpu.VMEM((_GCHUNK, H), jnp.bfloat16),
            pltpu.SemaphoreType.DMA,
        ],
    )
    def stage3(x_hbm, srctok_hbm, out_hbm, idx_v, rows0_v, rows1_v, sem):
        wid = lax.axis_index("s") * NC + lax.axis_index("c")
        base = wid * _ROWS_PW
        pltpu.sync_copy(srctok_hbm.at[pl.ds(base, _ROWS_PW)], idx_v)
        h0 = pltpu.async_copy(x_hbm.at[idx_v.at[pl.ds(0, _GCHUNK)]], rows0_v, sem)
        h1 = pltpu.async_copy(x_hbm.at[idx_v.at[pl.ds(_GCHUNK, _GCHUNK)]], rows1_v, sem)
        h0.wait()
        pltpu.sync_copy(rows0_v, out_hbm.at[pl.ds(base, _GCHUNK)])
        h1.wait()
        pltpu.sync_copy(rows1_v, out_hbm.at[pl.ds(base + _GCHUNK, _GCHUNK)])

    @functools.partial(
        pl.kernel,
        out_type=jax.ShapeDtypeStruct((S, H), jnp.float32),
        mesh=mesh,
        compiler_params=pltpu.CompilerParams(needs_layout_passes=False),
        scratch_types=[
            pltpu.VMEM((2 * _TOK_PW,), jnp.int32),
            pltpu.VMEM((2 * _TOK_PW,), jnp.float32),
            pltpu.VMEM((_TOK_PW,), jnp.int32),
            pltpu.VMEM((_TOK_PW,), jnp.int32),
            pltpu.VMEM((_TOK_PW, H), jnp.float32),
            pltpu.VMEM((_TOK_PW, H), jnp.float32),
            pltpu.SemaphoreType.DMA,
        ],
    )
    def stage5(y_hbm, dst_hbm, sc_hbm, out_hbm, dst_v, sc_v, i0_v, i1_v,
               b0_v, b1_v, sem):
        wid = lax.axis_index("s") * NC + lax.axis_index("c")
        base = wid * _TOK_PW
        pltpu.sync_copy(dst_hbm.at[pl.ds(base * 2, 2 * _TOK_PW)], dst_v)
        pltpu.sync_copy(sc_hbm.at[pl.ds(base * 2, 2 * _TOK_PW)], sc_v)

        for z in range(_TOK_PW // SC_L):
            g = (z * SC_L + lax.iota(jnp.int32, SC_L)) * 2
            i0_v[pl.ds(z * SC_L, SC_L)] = plsc.load_gather(dst_v, [g])
            i1_v[pl.ds(z * SC_L, SC_L)] = plsc.load_gather(dst_v, [g + 1])

        pltpu.async_copy(y_hbm.at[i0_v], b0_v, sem).wait()
        pltpu.async_copy(y_hbm.at[i1_v], b1_v, sem).wait()

        def row_step(i, _):
            s0 = plsc.load_gather(sc_v, [jnp.full((SC_L,), 2 * i, jnp.int32)])
            s1 = plsc.load_gather(sc_v, [jnp.full((SC_L,), 2 * i + 1, jnp.int32)])
            for h in range(H // SC_L):
                sl = pl.ds(h * SC_L, SC_L)
                b0_v[i, sl] = s0 * b0_v[i, sl] + s1 * b1_v[i, sl]
            return 0

        lax.fori_loop(0, _TOK_PW, row_step, 0)
        pltpu.sync_copy(b0_v, out_hbm.at[pl.ds(base, _TOK_PW)])

    return stage2, stage3, stage5


def _mlp_body(be_ref, x_ref, wgu_ref, bgu_ref, wd_ref, bd_ref, y_ref):
    x = x_ref[...]
    gu = jnp.dot(x, wgu_ref[0].astype(jnp.bfloat16),
                 preferred_element_type=jnp.float32)
    gu = gu + bgu_ref[0]
    gu = jnp.minimum(gu, LIMIT)
    gate = gu[:, :I]
    up = jnp.maximum(gu[:, I:], -LIMIT)
    glu = gate * jax.nn.sigmoid(gate * ALPHA)
    act = ((up + 1.0) * glu).astype(jnp.bfloat16)
    y = jnp.dot(act, wd_ref[0].astype(jnp.bfloat16),
                preferred_element_type=jnp.float32)
    y_ref[...] = y + bd_ref[0]


def _stage4(be, xs, wgu, bgu, wd, bd):
    grid_spec = pltpu.PrefetchScalarGridSpec(
        num_scalar_prefetch=1,
        grid=(NB,),
        in_specs=[
            pl.BlockSpec((BLK, H), lambda v, be: (v, 0)),
            pl.BlockSpec((1, H, 2 * I), lambda v, be: (be[v], 0, 0)),
            pl.BlockSpec((1, 1, 2 * I), lambda v, be: (be[v], 0, 0)),
            pl.BlockSpec((1, I, H), lambda v, be: (be[v], 0, 0)),
            pl.BlockSpec((1, 1, H), lambda v, be: (be[v], 0, 0)),
        ],
        out_specs=pl.BlockSpec((BLK, H), lambda v, be: (v, 0)),
    )
    return pl.pallas_call(
        _mlp_body,
        grid_spec=grid_spec,
        out_shape=jax.ShapeDtypeStruct((CAP, H), jnp.float32),
    )(be, xs, wgu, bgu.reshape(E, 1, 2 * I), wd, bd.reshape(E, 1, H))


def kernel(hidden_states, router_w, router_b, gate_up_proj, gate_up_proj_bias,
           down_proj, down_proj_bias):
    B = hidden_states.shape[0]
    x2 = hidden_states.reshape(S, H)
    w_pad = jnp.pad(router_w, ((0, 0), (0, LANES - E)))
    b_pad = jnp.pad(router_b, (0, LANES - E)).reshape(1, LANES)

    scores_pad, dst_pad, be_pad = _stage1(x2, w_pad, b_pad)
    scores2 = scores_pad[:, :K]
    dst2 = dst_pad[:, :K]
    be = be_pad[:, 0]
    dst_flat = dst2.reshape(_PAIRS)

    stage2, stage3, stage5 = _sc_stages()
    src_tok = stage2(dst_flat)
    x_bf = hidden_states.astype(jnp.bfloat16).reshape(S, H)
    xs = stage3(x_bf, src_tok)
    ys = _stage4(be, xs, gate_up_proj, gate_up_proj_bias,
                 down_proj, down_proj_bias)
    out = stage5(ys, dst_flat, scores2.reshape(_PAIRS))

    return out.reshape(B, S, H), scores2.reshape(B, S, K)


# speedup vs baseline: 1.6510x; 1.1837x over previous
import functools

import jax
import jax.numpy as jnp
from jax import lax
from jax.experimental import pallas as pl
from jax.experimental.pallas import tpu as pltpu
from jax.experimental.pallas import tpu_sc as plsc

S = 2048
H = 768
I = 768
E = 8
K = 2
ALPHA = 1.702
LIMIT = 7.0

BLK = 128
CAP = S * K + E * BLK
NB = CAP // BLK
LANES = 128

NC = 2
NS = 16
NW = NC * NS
SC_L = 16

NEG = -1e30


def _router_body(x_ref, w_ref, b_ref, scores_ref, dst_ref, be_ref, ohs_ref, t_ref):
    x = x_ref[...]
    w = w_ref[...]
    logits = jnp.dot(x, w, preferred_element_type=jnp.float32) + b_ref[...]
    lane = lax.broadcasted_iota(jnp.int32, (S, LANES), 1)
    valid = lane < E
    logits = jnp.where(valid, logits, NEG)

    m1 = jnp.max(logits, axis=1, keepdims=True)
    i1 = jnp.min(jnp.where(logits == m1, lane, LANES), axis=1, keepdims=True)
    l2 = jnp.where(lane == i1, NEG, logits)
    m2 = jnp.max(l2, axis=1, keepdims=True)
    i2 = jnp.min(jnp.where(l2 == m2, lane, LANES), axis=1, keepdims=True)

    t = jnp.exp(m2 - m1)
    s1 = 1.0 / (1.0 + t)
    s2 = t / (1.0 + t)
    scores_ref[...] = jnp.where(lane == 0, s1, jnp.where(lane == 1, s2, 0.0))

    onehot0 = (lane == i1).astype(jnp.float32)
    onehot1 = (lane == i2).astype(jnp.float32)
    ohs_ref[...] = onehot0 + onehot1

    r_sub = lax.broadcasted_iota(jnp.int32, (BLK, BLK), 0)
    r_lane = lax.broadcasted_iota(jnp.int32, (BLK, BLK), 1)
    ltri = (r_sub >= r_lane).astype(jnp.float32)

    def chunk_step(c, carry):
        chunk = ohs_ref[pl.ds(c * BLK, BLK), :]
        tc = jnp.dot(ltri, chunk, preferred_element_type=jnp.float32) + carry
        t_ref[pl.ds(c * BLK, BLK), :] = tc
        return tc[BLK - 1 : BLK, :]

    carry0 = jnp.zeros((1, LANES), jnp.float32)
    lax.fori_loop(0, S // BLK, chunk_step, carry0)

    counts = t_ref[S - 1 : S, :]
    aligned = jnp.floor((counts + (BLK - 1)) / BLK) * BLK
    stri = (r_sub < r_lane).astype(jnp.float32)
    off = jnp.dot(aligned, stri, preferred_element_type=jnp.float32)
    ends = off + aligned

    tfull = t_ref[...]
    dst0 = jnp.sum(onehot0 * (off + tfull), axis=1, keepdims=True) - 1.0
    dst1 = jnp.sum(onehot1 * (off + tfull), axis=1, keepdims=True) - 1.0
    dsts = jnp.where(lane == 0, dst0, jnp.where(lane == 1, dst1, 0.0))
    dst_ref[...] = dsts.astype(jnp.int32)

    v_sub = lax.broadcasted_iota(jnp.int32, (NB, LANES), 0).astype(jnp.float32) * BLK
    be_lane = lax.broadcasted_iota(jnp.int32, (NB, LANES), 1)
    ind = ((v_sub >= ends) & (be_lane < E)).astype(jnp.int32)
    be = jnp.minimum(jnp.sum(ind, axis=1, keepdims=True), E - 1)
    be_ref[...] = jnp.broadcast_to(be, (NB, LANES))


def _stage1(x, w_pad, b_pad):
    return pl.pallas_call(
        _router_body,
        out_shape=(
            jax.ShapeDtypeStruct((S, LANES), jnp.float32),
            jax.ShapeDtypeStruct((S, LANES), jnp.int32),
            jax.ShapeDtypeStruct((NB, LANES), jnp.int32),
        ),
        scratch_shapes=[
            pltpu.VMEM((S, LANES), jnp.float32),
            pltpu.VMEM((S, LANES), jnp.float32),
        ],
    )(x, w_pad, b_pad)


_SLOTS_PW = CAP // NW
_PAIRS = S * K
_ROWS_PW = CAP // NW
_GCHUNK = 80
_TOK_PW = S // NW


@functools.cache
def _sc_stages():
    mesh = plsc.VectorSubcoreMesh(
        core_axis_name="c", subcore_axis_name="s", num_cores=NC, num_subcores=NS
    )

    @functools.partial(
        pl.kernel,
        out_type=jax.ShapeDtypeStruct((CAP,), jnp.int32),
        mesh=mesh,
        compiler_params=pltpu.CompilerParams(needs_layout_passes=False),
        scratch_types=[
            pltpu.VMEM((_PAIRS,), jnp.int32),
            pltpu.VMEM((2 * BLK,), jnp.int32),
        ],
    )
    def stage2(dst_hbm, out_hbm, dst_v, st_v):
        wid = lax.axis_index("s") * NC + lax.axis_index("c")
        base = wid * _SLOTS_PW
        pltpu.sync_copy(dst_hbm, dst_v)

        for z in range(_SLOTS_PW // SC_L):
            st_v[pl.ds(z * SC_L, SC_L)] = jnp.zeros((SC_L,), jnp.int32)

        def step(c, _):
            idx = dst_v[pl.ds(c * SC_L, SC_L)]
            tok = (c * SC_L + lax.iota(jnp.int32, SC_L)) >> 1
            rel = idx - base
            mask = (rel >= 0) & (rel < _SLOTS_PW)
            relc = jnp.where(mask, rel, 0)
            plsc.store_scatter(st_v, [relc], tok, mask=mask)
            return 0

        lax.fori_loop(0, _PAIRS // SC_L, step, 0)
        pltpu.sync_copy(st_v.at[pl.ds(0, _SLOTS_PW)],
                        out_hbm.at[pl.ds(base, _SLOTS_PW)])

    @functools.partial(
        pl.kernel,
        out_type=jax.ShapeDtypeStruct((CAP, H), jnp.float32),
        mesh=mesh,
        compiler_params=pltpu.CompilerParams(needs_layout_passes=False),
        scratch_types=[
            pltpu.VMEM((_ROWS_PW,), jnp.int32),
            pltpu.VMEM((_GCHUNK, H), jnp.float32),
            pltpu.VMEM((_GCHUNK, H), jnp.float32),
            pltpu.SemaphoreType.DMA,
        ],
    )
    def stage3(x_hbm, srctok_hbm, out_hbm, idx_v, rows0_v, rows1_v, sem):
        wid = lax.axis_index("s") * NC + lax.axis_index("c")
        base = wid * _ROWS_PW
        pltpu.sync_copy(srctok_hbm.at[pl.ds(base, _ROWS_PW)], idx_v)
        h0 = pltpu.async_copy(x_hbm.at[idx_v.at[pl.ds(0, _GCHUNK)]], rows0_v, sem)
        h1 = pltpu.async_copy(x_hbm.at[idx_v.at[pl.ds(_GCHUNK, _GCHUNK)]], rows1_v, sem)
        h0.wait()
        pltpu.sync_copy(rows0_v, out_hbm.at[pl.ds(base, _GCHUNK)])
        h1.wait()
        pltpu.sync_copy(rows1_v, out_hbm.at[pl.ds(base + _GCHUNK, _GCHUNK)])

    @functools.partial(
        pl.kernel,
        out_type=jax.ShapeDtypeStruct((S, H), jnp.float32),
        mesh=mesh,
        compiler_params=pltpu.CompilerParams(needs_layout_passes=False),
        scratch_types=[
            pltpu.VMEM((2 * _TOK_PW,), jnp.int32),
            pltpu.VMEM((2 * _TOK_PW,), jnp.float32),
            pltpu.VMEM((_TOK_PW,), jnp.int32),
            pltpu.VMEM((_TOK_PW,), jnp.int32),
            pltpu.VMEM((_TOK_PW, H), jnp.float32),
            pltpu.VMEM((_TOK_PW, H), jnp.float32),
            pltpu.SemaphoreType.DMA,
        ],
    )
    def stage5(y_hbm, dst_hbm, sc_hbm, out_hbm, dst_v, sc_v, i0_v, i1_v,
               b0_v, b1_v, sem):
        wid = lax.axis_index("s") * NC + lax.axis_index("c")
        base = wid * _TOK_PW
        pltpu.sync_copy(dst_hbm.at[pl.ds(base * 2, 2 * _TOK_PW)], dst_v)
        pltpu.sync_copy(sc_hbm.at[pl.ds(base * 2, 2 * _TOK_PW)], sc_v)

        for z in range(_TOK_PW // SC_L):
            g = (z * SC_L + lax.iota(jnp.int32, SC_L)) * 2
            i0_v[pl.ds(z * SC_L, SC_L)] = plsc.load_gather(dst_v, [g])
            i1_v[pl.ds(z * SC_L, SC_L)] = plsc.load_gather(dst_v, [g + 1])

        pltpu.async_copy(y_hbm.at[i0_v], b0_v, sem).wait()
        pltpu.async_copy(y_hbm.at[i1_v], b1_v, sem).wait()

        def row_step(i, _):
            s0 = plsc.load_gather(sc_v, [jnp.full((SC_L,), 2 * i, jnp.int32)])
            s1 = plsc.load_gather(sc_v, [jnp.full((SC_L,), 2 * i + 1, jnp.int32)])
            for h in range(H // SC_L):
                sl = pl.ds(h * SC_L, SC_L)
                b0_v[i, sl] = s0 * b0_v[i, sl] + s1 * b1_v[i, sl]
            return 0

        lax.fori_loop(0, _TOK_PW, row_step, 0)
        pltpu.sync_copy(b0_v, out_hbm.at[pl.ds(base, _TOK_PW)])

    return stage2, stage3, stage5


def _mlp_body(be_ref, x_ref, wgu_ref, bgu_ref, wd_ref, bd_ref, y_ref,
              wgu_bf, wd_bf):
    v = pl.program_id(0)
    changed = (v == 0) | (be_ref[v] != be_ref[jnp.maximum(v - 1, 0)])

    @pl.when(changed)
    def _cast_weights():
        wgu_bf[...] = wgu_ref[0].astype(jnp.bfloat16)
        wd_bf[...] = wd_ref[0].astype(jnp.bfloat16)

    x = x_ref[...].astype(jnp.bfloat16)
    gu = jnp.dot(x, wgu_bf[...], preferred_element_type=jnp.float32)
    gu = gu + bgu_ref[0]
    gu = jnp.minimum(gu, LIMIT)
    gate = gu[:, :I]
    up = jnp.maximum(gu[:, I:], -LIMIT)
    glu = gate * jax.nn.sigmoid(gate * ALPHA)
    act = ((up + 1.0) * glu).astype(jnp.bfloat16)
    y = jnp.dot(act, wd_bf[...], preferred_element_type=jnp.float32)
    y_ref[...] = y + bd_ref[0]


def _stage4(be, xs, wgu, bgu, wd, bd):
    grid_spec = pltpu.PrefetchScalarGridSpec(
        num_scalar_prefetch=1,
        grid=(NB,),
        in_specs=[
            pl.BlockSpec((BLK, H), lambda v, be: (v, 0)),
            pl.BlockSpec((1, H, 2 * I), lambda v, be: (be[v], 0, 0)),
            pl.BlockSpec((1, 1, 2 * I), lambda v, be: (be[v], 0, 0)),
            pl.BlockSpec((1, I, H), lambda v, be: (be[v], 0, 0)),
            pl.BlockSpec((1, 1, H), lambda v, be: (be[v], 0, 0)),
        ],
        out_specs=pl.BlockSpec((BLK, H), lambda v, be: (v, 0)),
        scratch_shapes=[
            pltpu.VMEM((H, 2 * I), jnp.bfloat16),
            pltpu.VMEM((I, H), jnp.bfloat16),
        ],
    )
    return pl.pallas_call(
        _mlp_body,
        grid_spec=grid_spec,
        out_shape=jax.ShapeDtypeStruct((CAP, H), jnp.float32),
    )(be, xs, wgu, bgu.reshape(E, 1, 2 * I), wd, bd.reshape(E, 1, H))


def kernel(hidden_states, router_w, router_b, gate_up_proj, gate_up_proj_bias,
           down_proj, down_proj_bias):
    B = hidden_states.shape[0]
    x2 = hidden_states.reshape(S, H)
    w_pad = jnp.pad(router_w, ((0, 0), (0, LANES - E)))
    b_pad = jnp.pad(router_b, (0, LANES - E)).reshape(1, LANES)

    scores_pad, dst_pad, be_pad = _stage1(x2, w_pad, b_pad)
    scores2 = scores_pad[:, :K]
    dst2 = dst_pad[:, :K]
    be = be_pad[:, 0]
    dst_flat = dst2.reshape(_PAIRS)

    stage2, stage3, stage5 = _sc_stages()
    src_tok = stage2(dst_flat)
    xs = stage3(x2, src_tok)
    ys = _stage4(be, xs, gate_up_proj, gate_up_proj_bias,
                 down_proj, down_proj_bias)
    out = stage5(ys, dst_flat, scores2.reshape(_PAIRS))

    return out.reshape(B, S, H), scores2.reshape(B, S, K)


# speedup vs baseline: 1.7363x; 1.0516x over previous
import functools

import jax
import jax.numpy as jnp
from jax import lax
from jax.experimental import pallas as pl
from jax.experimental.pallas import tpu as pltpu
from jax.experimental.pallas import tpu_sc as plsc

S = 2048
H = 768
I = 768
E = 8
K = 2
ALPHA = 1.702
LIMIT = 7.0

BLK = 128
CAP = S * K + E * BLK
NB = CAP // BLK
LANES = 128

NC = 2
NS = 16
NW = NC * NS
SC_L = 16

NEG = -1e30


def _router_body(x_ref, w_ref, b_ref, scores_ref, dst_ref, be_ref, ohs_ref, t_ref):
    x = x_ref[...]
    w = w_ref[...]
    logits = jnp.dot(x, w, preferred_element_type=jnp.float32) + b_ref[...]
    lane = lax.broadcasted_iota(jnp.int32, (S, LANES), 1)
    valid = lane < E
    logits = jnp.where(valid, logits, NEG)

    m1 = jnp.max(logits, axis=1, keepdims=True)
    i1 = jnp.min(jnp.where(logits == m1, lane, LANES), axis=1, keepdims=True)
    l2 = jnp.where(lane == i1, NEG, logits)
    m2 = jnp.max(l2, axis=1, keepdims=True)
    i2 = jnp.min(jnp.where(l2 == m2, lane, LANES), axis=1, keepdims=True)

    t = jnp.exp(m2 - m1)
    s1 = 1.0 / (1.0 + t)
    s2 = t / (1.0 + t)
    scores_ref[...] = jnp.where(lane == 0, s1, jnp.where(lane == 1, s2, 0.0))

    onehot0 = (lane == i1).astype(jnp.float32)
    onehot1 = (lane == i2).astype(jnp.float32)
    ohs_ref[...] = onehot0 + onehot1

    r_sub = lax.broadcasted_iota(jnp.int32, (BLK, BLK), 0)
    r_lane = lax.broadcasted_iota(jnp.int32, (BLK, BLK), 1)
    ltri = (r_sub >= r_lane).astype(jnp.float32)

    def chunk_step(c, carry):
        chunk = ohs_ref[pl.ds(c * BLK, BLK), :]
        tc = jnp.dot(ltri, chunk, preferred_element_type=jnp.float32) + carry
        t_ref[pl.ds(c * BLK, BLK), :] = tc
        return tc[BLK - 1 : BLK, :]

    carry0 = jnp.zeros((1, LANES), jnp.float32)
    lax.fori_loop(0, S // BLK, chunk_step, carry0)

    counts = t_ref[S - 1 : S, :]
    aligned = jnp.floor((counts + (BLK - 1)) / BLK) * BLK
    stri = (r_sub < r_lane).astype(jnp.float32)
    off = jnp.dot(aligned, stri, preferred_element_type=jnp.float32)
    ends = off + aligned

    tfull = t_ref[...]
    dst0 = jnp.sum(onehot0 * (off + tfull), axis=1, keepdims=True) - 1.0
    dst1 = jnp.sum(onehot1 * (off + tfull), axis=1, keepdims=True) - 1.0
    dsts = jnp.where(lane == 0, dst0, jnp.where(lane == 1, dst1, 0.0))
    dst_ref[...] = dsts.astype(jnp.int32)

    v_sub = lax.broadcasted_iota(jnp.int32, (NB, LANES), 0).astype(jnp.float32) * BLK
    be_lane = lax.broadcasted_iota(jnp.int32, (NB, LANES), 1)
    ind = ((v_sub >= ends) & (be_lane < E)).astype(jnp.int32)
    be = jnp.minimum(jnp.sum(ind, axis=1, keepdims=True), E - 1)
    be_ref[...] = jnp.broadcast_to(be, (NB, LANES))


def _stage1(x, w_pad, b_pad):
    return pl.pallas_call(
        _router_body,
        out_shape=(
            jax.ShapeDtypeStruct((S, LANES), jnp.float32),
            jax.ShapeDtypeStruct((S, LANES), jnp.int32),
            jax.ShapeDtypeStruct((NB, LANES), jnp.int32),
        ),
        scratch_shapes=[
            pltpu.VMEM((S, LANES), jnp.float32),
            pltpu.VMEM((S, LANES), jnp.float32),
        ],
    )(x, w_pad, b_pad)


_SLOTS_PW = CAP // NW
_PAIRS = S * K
_ROWS_PW = CAP // NW
_GCHUNK = 80
_HALF_ROWS = CAP // 2
_HROWS_PW = _HALF_ROWS // NW
_TOK_PW = S // NW


@functools.cache
def _sc_stages():
    mesh = plsc.VectorSubcoreMesh(
        core_axis_name="c", subcore_axis_name="s", num_cores=NC, num_subcores=NS
    )

    @functools.partial(
        pl.kernel,
        out_type=jax.ShapeDtypeStruct((CAP,), jnp.int32),
        mesh=mesh,
        compiler_params=pltpu.CompilerParams(needs_layout_passes=False),
        scratch_types=[
            pltpu.VMEM((_PAIRS,), jnp.int32),
            pltpu.VMEM((2 * BLK,), jnp.int32),
        ],
    )
    def stage2(dst_hbm, out_hbm, dst_v, st_v):
        wid = lax.axis_index("s") * NC + lax.axis_index("c")
        base = wid * _SLOTS_PW
        pltpu.sync_copy(dst_hbm, dst_v)

        for z in range(_SLOTS_PW // SC_L):
            st_v[pl.ds(z * SC_L, SC_L)] = jnp.zeros((SC_L,), jnp.int32)

        def step(c, _):
            idx = dst_v[pl.ds(c * SC_L, SC_L)]
            tok = (c * SC_L + lax.iota(jnp.int32, SC_L)) >> 1
            rel = idx - base
            mask = (rel >= 0) & (rel < _SLOTS_PW)
            relc = jnp.where(mask, rel, 0)
            plsc.store_scatter(st_v, [relc], tok, mask=mask)
            return 0

        lax.fori_loop(0, _PAIRS // SC_L, step, 0)
        pltpu.sync_copy(st_v.at[pl.ds(0, _SLOTS_PW)],
                        out_hbm.at[pl.ds(base, _SLOTS_PW)])

    def make_stage3(h):
        @functools.partial(
            pl.kernel,
            out_type=jax.ShapeDtypeStruct((_HALF_ROWS, H), jnp.float32),
            mesh=mesh,
            compiler_params=pltpu.CompilerParams(needs_layout_passes=False),
            scratch_types=[
                pltpu.VMEM((_HROWS_PW,), jnp.int32),
                pltpu.VMEM((_HROWS_PW, H), jnp.float32),
                pltpu.SemaphoreType.DMA,
            ],
            name=f"gather_half{h}",
        )
        def stage3h(x_hbm, srctok_hbm, out_hbm, idx_v, rows_v, sem):
            wid = lax.axis_index("s") * NC + lax.axis_index("c")
            base = wid * _HROWS_PW
            pltpu.sync_copy(
                srctok_hbm.at[pl.ds(h * _HALF_ROWS + base, _HROWS_PW)], idx_v)
            pltpu.async_copy(x_hbm.at[idx_v], rows_v, sem).wait()
            pltpu.sync_copy(rows_v, out_hbm.at[pl.ds(base, _HROWS_PW)])
        return stage3h

    stage3 = (make_stage3(0), make_stage3(1))

    @functools.partial(
        pl.kernel,
        out_type=jax.ShapeDtypeStruct((S, H), jnp.float32),
        mesh=mesh,
        compiler_params=pltpu.CompilerParams(needs_layout_passes=False),
        scratch_types=[
            pltpu.VMEM((2 * _TOK_PW,), jnp.int32),
            pltpu.VMEM((2 * _TOK_PW,), jnp.float32),
            pltpu.VMEM((_TOK_PW,), jnp.int32),
            pltpu.VMEM((_TOK_PW,), jnp.int32),
            pltpu.VMEM((_TOK_PW, H), jnp.float32),
            pltpu.VMEM((_TOK_PW, H), jnp.float32),
            pltpu.SemaphoreType.DMA,
        ],
    )
    def stage5(y_hbm, dst_hbm, sc_hbm, out_hbm, dst_v, sc_v, i0_v, i1_v,
               b0_v, b1_v, sem):
        wid = lax.axis_index("s") * NC + lax.axis_index("c")
        base = wid * _TOK_PW
        pltpu.sync_copy(dst_hbm.at[pl.ds(base * 2, 2 * _TOK_PW)], dst_v)
        pltpu.sync_copy(sc_hbm.at[pl.ds(base * 2, 2 * _TOK_PW)], sc_v)

        for z in range(_TOK_PW // SC_L):
            g = (z * SC_L + lax.iota(jnp.int32, SC_L)) * 2
            i0_v[pl.ds(z * SC_L, SC_L)] = plsc.load_gather(dst_v, [g])
            i1_v[pl.ds(z * SC_L, SC_L)] = plsc.load_gather(dst_v, [g + 1])

        pltpu.async_copy(y_hbm.at[i0_v], b0_v, sem).wait()
        pltpu.async_copy(y_hbm.at[i1_v], b1_v, sem).wait()

        def row_step(i, _):
            s0 = plsc.load_gather(sc_v, [jnp.full((SC_L,), 2 * i, jnp.int32)])
            s1 = plsc.load_gather(sc_v, [jnp.full((SC_L,), 2 * i + 1, jnp.int32)])
            for h in range(H // SC_L):
                sl = pl.ds(h * SC_L, SC_L)
                b0_v[i, sl] = s0 * b0_v[i, sl] + s1 * b1_v[i, sl]
            return 0

        lax.fori_loop(0, _TOK_PW, row_step, 0)
        pltpu.sync_copy(b0_v, out_hbm.at[pl.ds(base, _TOK_PW)])

    return stage2, stage3, stage5


def _mlp_body(be_ref, x_ref, wgu_ref, bgu_ref, wd_ref, bd_ref, y_ref,
              wgu_bf, wd_bf):
    v = pl.program_id(0)
    changed = (v == 0) | (be_ref[v] != be_ref[jnp.maximum(v - 1, 0)])

    @pl.when(changed)
    def _cast_weights():
        wgu_bf[...] = wgu_ref[0].astype(jnp.bfloat16)
        wd_bf[...] = wd_ref[0].astype(jnp.bfloat16)

    x = x_ref[...].astype(jnp.bfloat16)
    gu = jnp.dot(x, wgu_bf[...], preferred_element_type=jnp.float32)
    gu = gu + bgu_ref[0]
    gu = jnp.minimum(gu, LIMIT)
    gate = gu[:, :I]
    up = jnp.maximum(gu[:, I:], -LIMIT)
    glu = gate * jax.nn.sigmoid(gate * ALPHA)
    act = ((up + 1.0) * glu).astype(jnp.bfloat16)
    y = jnp.dot(act, wd_bf[...], preferred_element_type=jnp.float32)
    y_ref[...] = y + bd_ref[0]


def _stage4_part(be_part, xs_part, wgu, bgu, wd, bd, y_in, off):
    nblk = _HALF_ROWS // BLK
    in_specs = [
        pl.BlockSpec((BLK, H), lambda v, be: (v, 0)),
        pl.BlockSpec((1, H, 2 * I), lambda v, be: (be[v], 0, 0)),
        pl.BlockSpec((1, 1, 2 * I), lambda v, be: (be[v], 0, 0)),
        pl.BlockSpec((1, I, H), lambda v, be: (be[v], 0, 0)),
        pl.BlockSpec((1, 1, H), lambda v, be: (be[v], 0, 0)),
    ]
    args = [be_part, xs_part, wgu, bgu.reshape(E, 1, 2 * I), wd,
            bd.reshape(E, 1, H)]
    aliases = {}
    if y_in is not None:
        in_specs.append(pl.BlockSpec(memory_space=pl.ANY))
        args.append(y_in)
        aliases = {6: 0}
    grid_spec = pltpu.PrefetchScalarGridSpec(
        num_scalar_prefetch=1,
        grid=(nblk,),
        in_specs=in_specs,
        out_specs=pl.BlockSpec((BLK, H), lambda v, be: (v + off, 0)),
        scratch_shapes=[
            pltpu.VMEM((H, 2 * I), jnp.bfloat16),
            pltpu.VMEM((I, H), jnp.bfloat16),
        ],
    )
    body = _mlp_body if y_in is None else (
        lambda be_r, x_r, a_r, b_r, c_r, d_r, yin_r, y_r, s1, s2:
            _mlp_body(be_r, x_r, a_r, b_r, c_r, d_r, y_r, s1, s2))
    return pl.pallas_call(
        body,
        grid_spec=grid_spec,
        out_shape=jax.ShapeDtypeStruct((CAP, H), jnp.float32),
        input_output_aliases=aliases,
    )(*args)


def kernel(hidden_states, router_w, router_b, gate_up_proj, gate_up_proj_bias,
           down_proj, down_proj_bias):
    B = hidden_states.shape[0]
    x2 = hidden_states.reshape(S, H)
    w_pad = jnp.pad(router_w, ((0, 0), (0, LANES - E)))
    b_pad = jnp.pad(router_b, (0, LANES - E)).reshape(1, LANES)

    scores_pad, dst_pad, be_pad = _stage1(x2, w_pad, b_pad)
    scores2 = scores_pad[:, :K]
    dst2 = dst_pad[:, :K]
    be = be_pad[:, 0]
    dst_flat = dst2.reshape(_PAIRS)

    stage2, (stage3a, stage3b), stage5 = _sc_stages()
    src_tok = stage2(dst_flat)
    nblk_half = _HALF_ROWS // BLK
    xs_a = stage3a(x2, src_tok)
    xs_b = stage3b(x2, src_tok)
    ys_a = _stage4_part(be[:nblk_half], xs_a, gate_up_proj, gate_up_proj_bias,
                        down_proj, down_proj_bias, None, 0)
    ys = _stage4_part(be[nblk_half:], xs_b, gate_up_proj, gate_up_proj_bias,
                      down_proj, down_proj_bias, ys_a, nblk_half)
    out = stage5(ys, dst_flat, scores2.reshape(_PAIRS))

    return out.reshape(B, S, H), scores2.reshape(B, S, K)
